# Initial kernel scaffold; baseline (speedup 1.0000x reference)
import jax
import jax.numpy as jnp
from jax.experimental import pallas as pl


def kernel(x, adj_m, W0, b0, g0, be0, W1, b1, g1, be1, W2, b2, g2, be2, Wo, bo):
    raise NotImplementedError("write your pallas kernel here")


# speedup vs baseline: 18.3086x; 18.3086x over previous
import functools

import jax
import jax.numpy as jnp
from jax import lax
from jax.experimental import pallas as pl
from jax.experimental.pallas import tpu as pltpu
from jax.experimental.pallas import tpu_sc as plsc

N = 10000
E = 320000
D = 128
EPS = 1e-5

NC = 2
NS = 16
NW = NC * NS
EPW = E // NW
CHUNK = 125
NCHUNK = EPW // CHUNK
ROWS_A = 624
ZROWS = 104
REM = N - NS * ROWS_A
DEGW = 16


@functools.cache
def _make_deg_kernel():
    return functools.partial(
        pl.kernel,
        mesh=plsc.VectorSubcoreMesh(core_axis_name="c", subcore_axis_name="s"),
        compiler_params=pltpu.CompilerParams(use_tc_tiling_on_sc=False),
        out_type=jax.ShapeDtypeStruct((NC, N, DEGW), jnp.float32),
        scratch_types=[
            pltpu.VMEM((NCHUNK, CHUNK), jnp.int32),
            pltpu.VMEM((CHUNK, DEGW), jnp.float32),
            pltpu.VMEM((ROWS_A, DEGW), jnp.float32),
            pltpu.VMEM_SHARED((N, DEGW), jnp.float32),
        ],
    )(_deg_body)


def _deg_body(dst_hbm, out_hbm, dstidx_v, ones_v, zero_v, deg_sh):
    c = lax.axis_index("c")
    s = lax.axis_index("s")
    w = c * NS + s

    def fill(i, carry):
        ones_v[i, pl.ds(0, 16)] = jnp.ones((16,), jnp.float32)
        return carry

    lax.fori_loop(0, CHUNK, fill, 0)

    def zrow(i, carry):
        zero_v[i, pl.ds(0, 16)] = jnp.zeros((16,), jnp.float32)
        return carry

    lax.fori_loop(0, ROWS_A, zrow, 0)
    pltpu.sync_copy(zero_v, deg_sh.at[pl.ds(s * ROWS_A, ROWS_A)])

    @pl.when(s == 0)
    def _():
        pltpu.sync_copy(zero_v.at[pl.ds(0, REM)],
                        deg_sh.at[pl.ds(NS * ROWS_A, REM)])

    pltpu.sync_copy(dst_hbm.at[pl.ds(w * NCHUNK, NCHUNK)], dstidx_v)
    plsc.subcore_barrier()

    def body(j, carry):
        pltpu.sync_copy(ones_v, deg_sh.at[dstidx_v.at[j]], add=True)
        return carry

    lax.fori_loop(0, NCHUNK, body, 0)
    plsc.subcore_barrier()
    pltpu.sync_copy(deg_sh.at[pl.ds(s * ROWS_A, ROWS_A)],
                    out_hbm.at[c].at[pl.ds(s * ROWS_A, ROWS_A)])

    @pl.when(s == 0)
    def _():
        pltpu.sync_copy(deg_sh.at[pl.ds(NS * ROWS_A, REM)],
                        out_hbm.at[c].at[pl.ds(NS * ROWS_A, REM)])


@functools.cache
def _make_edge_kernel():
    return functools.partial(
        pl.kernel,
        mesh=plsc.VectorSubcoreMesh(core_axis_name="c", subcore_axis_name="s"),
        compiler_params=pltpu.CompilerParams(use_tc_tiling_on_sc=False),
        out_type=jax.ShapeDtypeStruct((NC, N, D), jnp.float32),
        scratch_types=[
            pltpu.VMEM((NCHUNK, CHUNK), jnp.int32),
            pltpu.VMEM((NCHUNK, CHUNK), jnp.int32),
            pltpu.VMEM((CHUNK, D), jnp.float32),
            pltpu.VMEM((ZROWS, D), jnp.float32),
            pltpu.VMEM_SHARED((N, D), jnp.float32),
            pltpu.SemaphoreType.DMA,
        ],
    )(_edge_body)


def _edge_body(h2_hbm, src_hbm, dst_hbm, out_hbm,
               srcidx_v, dstidx_v, rows_v, zero_v, acc_sh, sem):
    c = lax.axis_index("c")
    s = lax.axis_index("s")
    w = c * NS + s

    def zrow(i, carry):
        for j in range(D // 16):
            zero_v[i, pl.ds(j * 16, 16)] = jnp.zeros((16,), jnp.float32)
        return carry

    lax.fori_loop(0, ZROWS, zrow, 0)

    def zacc(k, carry):
        pltpu.sync_copy(zero_v, acc_sh.at[pl.ds(s * ROWS_A + k * ZROWS, ZROWS)])
        return carry

    lax.fori_loop(0, ROWS_A // ZROWS, zacc, 0)

    @pl.when(s == 0)
    def _():
        pltpu.sync_copy(zero_v.at[pl.ds(0, REM)],
                        acc_sh.at[pl.ds(NS * ROWS_A, REM)])

    pltpu.sync_copy(src_hbm.at[pl.ds(w * NCHUNK, NCHUNK)], srcidx_v)
    pltpu.sync_copy(dst_hbm.at[pl.ds(w * NCHUNK, NCHUNK)], dstidx_v)
    plsc.subcore_barrier()

    def body(j, carry):
        pltpu.async_copy(h2_hbm.at[srcidx_v.at[j]], rows_v, sem).wait()
        pltpu.sync_copy(rows_v, acc_sh.at[dstidx_v.at[j]], add=True)
        return carry

    lax.fori_loop(0, NCHUNK, body, 0)
    plsc.subcore_barrier()
    pltpu.sync_copy(acc_sh.at[pl.ds(s * ROWS_A, ROWS_A)],
                    out_hbm.at[c].at[pl.ds(s * ROWS_A, ROWS_A)])

    @pl.when(s == 0)
    def _():
        pltpu.sync_copy(acc_sh.at[pl.ds(NS * ROWS_A, REM)],
                        out_hbm.at[c].at[pl.ds(NS * ROWS_A, REM)])


def _t0_body(x_ref, w_ref, dega_ref, degb_ref, h2_ref, dinv_ref):
    deg = dega_ref[:, :1] + degb_ref[:, :1] + 1.0
    dinv = lax.rsqrt(jnp.maximum(deg, 1.0))
    dinv_ref[...] = dinv
    h = jnp.dot(x_ref[...], w_ref[...], preferred_element_type=jnp.float32)
    h2_ref[...] = h * dinv


def _mid_body(acca_ref, accb_ref, h2p_ref, dinv_ref, b_ref, g_ref, be_ref,
              w_ref, hact_ref, h2_ref):
    dinv = dinv_ref[...]
    t = (acca_ref[...] + accb_ref[...] + h2p_ref[...]) * dinv + b_ref[...]
    mean = jnp.mean(t, axis=0, keepdims=True)
    var = jnp.mean((t - mean) ** 2, axis=0, keepdims=True)
    hact = jnp.maximum((t - mean) * lax.rsqrt(var + EPS) * g_ref[...]
                       + be_ref[...], 0.0)
    hact_ref[...] = hact
    h2_ref[...] = jnp.dot(hact, w_ref[...],
                          preferred_element_type=jnp.float32) * dinv


def _fin_body(acca_ref, accb_ref, h2p_ref, dinv_ref, b_ref, g_ref, be_ref,
              hact1_ref, hact2_ref, wo_ref, bo_ref, out_ref):
    dinv = dinv_ref[...]
    t = (acca_ref[...] + accb_ref[...] + h2p_ref[...]) * dinv + b_ref[...]
    mean = jnp.mean(t, axis=0, keepdims=True)
    var = jnp.mean((t - mean) ** 2, axis=0, keepdims=True)
    hact3 = jnp.maximum((t - mean) * lax.rsqrt(var + EPS) * g_ref[...]
                        + be_ref[...], 0.0)
    hj = jnp.maximum(jnp.maximum(hact1_ref[...], hact2_ref[...]), hact3)
    o = jnp.dot(hj, wo_ref[...], preferred_element_type=jnp.float32) + bo_ref[...]
    m = jnp.max(o, axis=1, keepdims=True)
    sh = o - m
    lse = jnp.log(jnp.sum(jnp.exp(sh), axis=1, keepdims=True))
    out_ref[...] = sh - lse


_t0_call = pl.pallas_call(
    _t0_body,
    out_shape=[jax.ShapeDtypeStruct((N, D), jnp.float32),
               jax.ShapeDtypeStruct((N, 1), jnp.float32)],
)

_mid_call = pl.pallas_call(
    _mid_body,
    out_shape=[jax.ShapeDtypeStruct((N, D), jnp.float32),
               jax.ShapeDtypeStruct((N, D), jnp.float32)],
)

_fin_call = pl.pallas_call(
    _fin_body,
    out_shape=jax.ShapeDtypeStruct((N, D), jnp.float32),
)


def kernel(x, adj_m, W0, b0, g0, be0, W1, b1, g1, be1, W2, b2, g2, be2, Wo, bo):
    src = adj_m[0].reshape(NW * NCHUNK, CHUNK)
    dst = adj_m[1].reshape(NW * NCHUNK, CHUNK)

    degp = _make_deg_kernel()(dst)
    h2, dinv = _t0_call(x, W0, degp[0], degp[1])

    r1 = lambda v: v.reshape(1, D)
    hacts = []
    for (b, g, be, Wn) in ((b0, g0, be0, W1), (b1, g1, be1, W2)):
        accp = _make_edge_kernel()(h2, src, dst)
        hact, h2 = _mid_call(accp[0], accp[1], h2, dinv,
                             r1(b), r1(g), r1(be), Wn)
        hacts.append(hact)

    accp = _make_edge_kernel()(h2, src, dst)
    return _fin_call(accp[0], accp[1], h2, dinv,
                     r1(b2), r1(g2), r1(be2), hacts[0], hacts[1], Wo, bo)


# speedup vs baseline: 25.6069x; 1.3986x over previous
import functools

import jax
import jax.numpy as jnp
from jax import lax
from jax.experimental import pallas as pl
from jax.experimental.pallas import tpu as pltpu
from jax.experimental.pallas import tpu_sc as plsc

N = 10000
E = 320000
D = 128
EPS = 1e-5

NC = 2
NS = 16
NW = NC * NS
EPW = E // NW
CHUNK = 100
NCHUNK = EPW // CHUNK
ROWS_A = 624
ZROWS = 104
REM = N - NS * ROWS_A
DEGW = 16


@functools.cache
def _make_deg_kernel():
    return functools.partial(
        pl.kernel,
        mesh=plsc.VectorSubcoreMesh(core_axis_name="c", subcore_axis_name="s"),
        compiler_params=pltpu.CompilerParams(use_tc_tiling_on_sc=False),
        out_type=jax.ShapeDtypeStruct((NC, N, DEGW), jnp.float32),
        scratch_types=[
            pltpu.VMEM((NCHUNK, CHUNK), jnp.int32),
            pltpu.VMEM((CHUNK, DEGW), jnp.float32),
            pltpu.VMEM((ROWS_A, DEGW), jnp.float32),
            pltpu.VMEM_SHARED((N, DEGW), jnp.float32),
        ],
    )(_deg_body)


def _deg_body(dst_hbm, out_hbm, dstidx_v, ones_v, zero_v, deg_sh):
    c = lax.axis_index("c")
    s = lax.axis_index("s")
    w = c * NS + s

    def fill(i, carry):
        ones_v[i, pl.ds(0, 16)] = jnp.ones((16,), jnp.float32)
        return carry

    lax.fori_loop(0, CHUNK, fill, 0)

    def zrow(i, carry):
        zero_v[i, pl.ds(0, 16)] = jnp.zeros((16,), jnp.float32)
        return carry

    lax.fori_loop(0, ROWS_A, zrow, 0)
    pltpu.sync_copy(zero_v, deg_sh.at[pl.ds(s * ROWS_A, ROWS_A)])

    @pl.when(s == 0)
    def _():
        pltpu.sync_copy(zero_v.at[pl.ds(0, REM)],
                        deg_sh.at[pl.ds(NS * ROWS_A, REM)])

    pltpu.sync_copy(dst_hbm.at[pl.ds(w * NCHUNK, NCHUNK)], dstidx_v)
    plsc.subcore_barrier()

    def body(j, carry):
        pltpu.sync_copy(ones_v, deg_sh.at[dstidx_v.at[j]], add=True)
        return carry

    lax.fori_loop(0, NCHUNK, body, 0)
    plsc.subcore_barrier()
    pltpu.sync_copy(deg_sh.at[pl.ds(s * ROWS_A, ROWS_A)],
                    out_hbm.at[c].at[pl.ds(s * ROWS_A, ROWS_A)])

    @pl.when(s == 0)
    def _():
        pltpu.sync_copy(deg_sh.at[pl.ds(NS * ROWS_A, REM)],
                        out_hbm.at[c].at[pl.ds(NS * ROWS_A, REM)])


@functools.cache
def _make_edge_kernel():
    return functools.partial(
        pl.kernel,
        mesh=plsc.VectorSubcoreMesh(core_axis_name="c", subcore_axis_name="s"),
        compiler_params=pltpu.CompilerParams(use_tc_tiling_on_sc=False),
        out_type=jax.ShapeDtypeStruct((NC, N, D), jnp.float32),
        scratch_types=[
            pltpu.VMEM((NCHUNK, CHUNK), jnp.int32),
            pltpu.VMEM((NCHUNK, CHUNK), jnp.int32),
            pltpu.VMEM((CHUNK, D), jnp.float32),
            pltpu.VMEM((CHUNK, D), jnp.float32),
            pltpu.VMEM_SHARED((N, D), jnp.float32),
            pltpu.SemaphoreType.DMA,
            pltpu.SemaphoreType.DMA,
        ],
    )(_edge_body)


NB = 2


def _edge_body(h2_hbm, src_hbm, dst_hbm, out_hbm,
               srcidx_v, dstidx_v, r0, r1, acc_sh, s0, s1):
    bufs = (r0, r1)
    sems = (s0, s1)
    c = lax.axis_index("c")
    s = lax.axis_index("s")
    w = c * NS + s

    def zrow(i, carry):
        for j in range(D // 16):
            r0[i, pl.ds(j * 16, 16)] = jnp.zeros((16,), jnp.float32)
        return carry

    lax.fori_loop(0, 96, zrow, 0)

    def zacc(k, carry):
        pltpu.sync_copy(r0.at[pl.ds(0, 96)],
                        acc_sh.at[pl.ds(s * ROWS_A + k * 96, 96)])
        return carry

    lax.fori_loop(0, 6, zacc, 0)
    pltpu.sync_copy(r0.at[pl.ds(0, 48)],
                    acc_sh.at[pl.ds(s * ROWS_A + 576, 48)])

    @pl.when(s == 0)
    def _():
        pltpu.sync_copy(r0.at[pl.ds(0, REM)],
                        acc_sh.at[pl.ds(NS * ROWS_A, REM)])

    pltpu.sync_copy(src_hbm.at[pl.ds(w * NCHUNK, NCHUNK)], srcidx_v)
    pltpu.sync_copy(dst_hbm.at[pl.ds(w * NCHUNK, NCHUNK)], dstidx_v)
    plsc.subcore_barrier()

    for b in range(NB):
        pltpu.async_copy(h2_hbm.at[srcidx_v.at[b]], bufs[b], sems[b])

    def body(g, carry):
        for b in range(NB):
            j = g * NB + b
            pltpu.make_async_copy(h2_hbm.at[srcidx_v.at[j]], bufs[b],
                                  sems[b]).wait()
            pltpu.sync_copy(bufs[b], acc_sh.at[dstidx_v.at[j]], add=True)

            @pl.when(j + NB < NCHUNK)
            def _():
                pltpu.async_copy(h2_hbm.at[srcidx_v.at[j + NB]],
                                 bufs[b], sems[b])
        return carry

    lax.fori_loop(0, NCHUNK // NB, body, 0)
    plsc.subcore_barrier()
    pltpu.sync_copy(acc_sh.at[pl.ds(s * ROWS_A, ROWS_A)],
                    out_hbm.at[c].at[pl.ds(s * ROWS_A, ROWS_A)])

    @pl.when(s == 0)
    def _():
        pltpu.sync_copy(acc_sh.at[pl.ds(NS * ROWS_A, REM)],
                        out_hbm.at[c].at[pl.ds(NS * ROWS_A, REM)])


def _t0_body(x_ref, w_ref, dega_ref, degb_ref, h2_ref, dinv_ref):
    deg = dega_ref[:, :1] + degb_ref[:, :1] + 1.0
    dinv = lax.rsqrt(jnp.maximum(deg, 1.0))
    dinv_ref[...] = dinv
    h = jnp.dot(x_ref[...], w_ref[...], preferred_element_type=jnp.float32)
    h2_ref[...] = h * dinv


def _mid_body(acca_ref, accb_ref, h2p_ref, dinv_ref, b_ref, g_ref, be_ref,
              w_ref, hact_ref, h2_ref):
    dinv = dinv_ref[...]
    t = (acca_ref[...] + accb_ref[...] + h2p_ref[...]) * dinv + b_ref[...]
    mean = jnp.mean(t, axis=0, keepdims=True)
    var = jnp.mean((t - mean) ** 2, axis=0, keepdims=True)
    hact = jnp.maximum((t - mean) * lax.rsqrt(var + EPS) * g_ref[...]
                       + be_ref[...], 0.0)
    hact_ref[...] = hact
    h2_ref[...] = jnp.dot(hact, w_ref[...],
                          preferred_element_type=jnp.float32) * dinv


def _fin_body(acca_ref, accb_ref, h2p_ref, dinv_ref, b_ref, g_ref, be_ref,
              hact1_ref, hact2_ref, wo_ref, bo_ref, out_ref):
    dinv = dinv_ref[...]
    t = (acca_ref[...] + accb_ref[...] + h2p_ref[...]) * dinv + b_ref[...]
    mean = jnp.mean(t, axis=0, keepdims=True)
    var = jnp.mean((t - mean) ** 2, axis=0, keepdims=True)
    hact3 = jnp.maximum((t - mean) * lax.rsqrt(var + EPS) * g_ref[...]
                        + be_ref[...], 0.0)
    hj = jnp.maximum(jnp.maximum(hact1_ref[...], hact2_ref[...]), hact3)
    o = jnp.dot(hj, wo_ref[...], preferred_element_type=jnp.float32) + bo_ref[...]
    m = jnp.max(o, axis=1, keepdims=True)
    sh = o - m
    lse = jnp.log(jnp.sum(jnp.exp(sh), axis=1, keepdims=True))
    out_ref[...] = sh - lse


_t0_call = pl.pallas_call(
    _t0_body,
    out_shape=[jax.ShapeDtypeStruct((N, D), jnp.float32),
               jax.ShapeDtypeStruct((N, 1), jnp.float32)],
)

_mid_call = pl.pallas_call(
    _mid_body,
    out_shape=[jax.ShapeDtypeStruct((N, D), jnp.float32),
               jax.ShapeDtypeStruct((N, D), jnp.float32)],
)

_fin_call = pl.pallas_call(
    _fin_body,
    out_shape=jax.ShapeDtypeStruct((N, D), jnp.float32),
)


def kernel(x, adj_m, W0, b0, g0, be0, W1, b1, g1, be1, W2, b2, g2, be2, Wo, bo):
    src = adj_m[0].reshape(NW * NCHUNK, CHUNK)
    dst = adj_m[1].reshape(NW * NCHUNK, CHUNK)

    degp = _make_deg_kernel()(dst)
    h2, dinv = _t0_call(x, W0, degp[0], degp[1])

    r1 = lambda v: v.reshape(1, D)
    hacts = []
    for (b, g, be, Wn) in ((b0, g0, be0, W1), (b1, g1, be1, W2)):
        accp = _make_edge_kernel()(h2, src, dst)
        hact, h2 = _mid_call(accp[0], accp[1], h2, dinv,
                             r1(b), r1(g), r1(be), Wn)
        hacts.append(hact)

    accp = _make_edge_kernel()(h2, src, dst)
    return _fin_call(accp[0], accp[1], h2, dinv,
                     r1(b2), r1(g2), r1(be2), hacts[0], hacts[1], Wo, bo)
